# Initial kernel scaffold; baseline (speedup 1.0000x reference)
import jax
import jax.numpy as jnp
from jax.experimental import pallas as pl


def kernel(x, edge_index, batch, emb_table, W1l, b1, W1r, W2l, b2, W2r, Wlin, blin):
    raise NotImplementedError("write your pallas kernel here")


# speedup vs baseline: 4.1820x; 4.1820x over previous
import functools

import jax
import jax.numpy as jnp
from jax import lax
from jax.experimental import pallas as pl
from jax.experimental.pallas import tpu as pltpu
from jax.experimental.pallas import tpu_sc as plsc

N = 50000
E = 800000
EMB = 64
HID = 64
NCLS = 2
G = 512

NTILE = 16
NCORE = 2
NP = 51200
EP = 819200
NACC = NP + 8
GP = 520

ESUB = 2
ECHUNK = ESUB * 128
EPT = EP // NTILE
NCHUNK = EPT // ECHUNK
ERPT = EPT // 128
RPT = NP // NTILE
WCH = 160
NPT = NP // NTILE

_f32 = jnp.float32
_mesh = plsc.VectorSubcoreMesh(core_axis_name="c", subcore_axis_name="s")


@functools.partial(
    pl.kernel,
    out_type=jax.ShapeDtypeStruct((NP, EMB), _f32),
    mesh=_mesh,
    scratch_types=[
        pltpu.VMEM((16, 100), jnp.int32),
        pltpu.VMEM((1600, EMB), _f32),
        pltpu.SemaphoreType.DMA,
    ],
    compiler_params=pltpu.CompilerParams(use_tc_tiling_on_sc=False),
)
def _emb_gather(x_r, emb, h0, idx_v, rows_v, sem):
    c = lax.axis_index("c")
    s = lax.axis_index("s")
    w = s * NCORE + c
    pltpu.sync_copy(x_r.at[pl.ds(w * 16, 16)], idx_v)
    descs = [
        pltpu.async_copy(emb.at[idx_v.at[j]], rows_v.at[pl.ds(j * 100, 100)], sem)
        for j in range(16)
    ]
    for d in descs:
        d.wait()
    pltpu.sync_copy(rows_v, h0.at[pl.ds(w * 1600, 1600)])


def _make_scatter(first_layer: bool):
    if first_layer:
        out_type = [
            jax.ShapeDtypeStruct((NP, 32), _f32),
            jax.ShapeDtypeStruct((NP, 32), _f32),
            jax.ShapeDtypeStruct((NP,), _f32),
        ]
    else:
        out_type = [
            jax.ShapeDtypeStruct((NP, 32), _f32),
            jax.ShapeDtypeStruct((NP, 32), _f32),
        ]

    scratch = [
        pltpu.VMEM_SHARED((NACC, 32), _f32),
        pltpu.VMEM_SHARED((NACC,), _f32),
        pltpu.VMEM((ESUB, 128), jnp.int32),
        pltpu.VMEM((ESUB, 128), jnp.int32),
        pltpu.VMEM((ECHUNK, 32), _f32),
        pltpu.VMEM((128,), _f32),
        pltpu.VMEM((WCH,), _f32),
        pltpu.SemaphoreType.DMA,
    ]

    def body(srcr, dstr, tabA, tabB, *rest):
        if first_layer:
            zrows, zflat, ones, outA, outB, invout = rest[:6]
            acc, cnt, src2d, dst2d, rows, ones_v, cbuf, sem = rest[6:]
        else:
            invin, zrows, zflat, ones, outA, outB = rest[:6]
            acc, cnt, src2d, dst2d, rows, ones_v, cbuf, sem = rest[6:]
        wbuf = rows.at[pl.ds(0, WCH)]

        c = lax.axis_index("c")
        t = lax.axis_index("s")

        pltpu.sync_copy(zrows, acc.at[pl.ds(t * RPT, RPT)])
        if first_layer:
            pltpu.sync_copy(zflat, cnt.at[pl.ds(t * RPT, RPT)])
            pltpu.sync_copy(ones, ones_v)
        plsc.subcore_barrier()

        def edge_loop(tab):
            def g_body(g, carry):
                r0 = t * ERPT + g * ESUB
                pltpu.sync_copy(srcr.at[pl.ds(r0, ESUB)], src2d)
                pltpu.sync_copy(dstr.at[pl.ds(r0, ESUB)], dst2d)
                descs = [
                    pltpu.async_copy(
                        tab.at[src2d.at[j]], rows.at[pl.ds(j * 128, 128)], sem
                    )
                    for j in range(ESUB)
                ]
                for d in descs:
                    d.wait()
                for j in range(ESUB):
                    pltpu.sync_copy(
                        rows.at[pl.ds(j * 128, 128)], acc.at[dst2d.at[j]], add=True
                    )
                    if first_layer:
                        pltpu.sync_copy(ones_v, cnt.at[dst2d.at[j]], add=True)
                return carry

            lax.fori_loop(0, NCHUNK, g_body, 0)

        @pl.when(c == 0)
        def _():
            edge_loop(tabA)

        @pl.when(c == 1)
        def _():
            edge_loop(tabB)

        plsc.subcore_barrier()

        def w_body(k, carry):
            r0 = t * RPT + k * WCH
            pltpu.sync_copy(acc.at[pl.ds(r0, WCH)], wbuf)
            if first_layer:
                pltpu.sync_copy(cnt.at[pl.ds(r0, WCH)], cbuf)
            else:
                pltpu.sync_copy(invin.at[pl.ds(r0, WCH)], cbuf)

            def r_body(r16, rc):
                base = r16 * 16
                cv = cbuf[pl.ds(base, 16)]
                if first_layer:
                    iv = 1.0 / jnp.maximum(cv, 1.0)
                    cbuf[pl.ds(base, 16)] = iv
                else:
                    iv = cv
                for i in range(16):
                    s = iv[i]
                    wbuf[base + i, pl.ds(0, 16)] = wbuf[base + i, pl.ds(0, 16)] * s
                    wbuf[base + i, pl.ds(16, 16)] = wbuf[base + i, pl.ds(16, 16)] * s
                return rc

            lax.fori_loop(0, WCH // 16, r_body, 0)

            @pl.when(c == 0)
            def _():
                pltpu.sync_copy(wbuf, outA.at[pl.ds(r0, WCH)])
                if first_layer:
                    pltpu.sync_copy(cbuf, invout.at[pl.ds(r0, WCH)])

            @pl.when(c == 1)
            def _():
                pltpu.sync_copy(wbuf, outB.at[pl.ds(r0, WCH)])

            return carry

        lax.fori_loop(0, RPT // WCH, w_body, 0)

    return functools.partial(
        pl.kernel, out_type=out_type, mesh=_mesh, scratch_types=scratch,
        compiler_params=pltpu.CompilerParams(use_tc_tiling_on_sc=False),
    )(body)


_scatter1 = _make_scatter(True)
_scatter2 = _make_scatter(False)


@functools.partial(
    pl.kernel,
    out_type=[
        jax.ShapeDtypeStruct((G, 32), _f32),
        jax.ShapeDtypeStruct((G, 32), _f32),
    ],
    mesh=_mesh,
    scratch_types=[
        pltpu.VMEM_SHARED((GP, 32), _f32),
        pltpu.VMEM_SHARED((GP,), _f32),
        pltpu.VMEM((NPT // 128, 128), jnp.int32),
        pltpu.VMEM((640, 32), _f32),
        pltpu.VMEM((128,), _f32),
        pltpu.VMEM((640,), _f32),
        pltpu.SemaphoreType.DMA,
    ],
    compiler_params=pltpu.CompilerParams(use_tc_tiling_on_sc=False),
)
def _pool(h2A, h2B, batch_r, zrows, zflat, ones, pA, pB, pacc, gcnt, bat2d,
          rows_v, ones_v, cbuf, sem):
    c = lax.axis_index("c")
    t = lax.axis_index("s")

    @pl.when(t == 0)
    def _():
        pltpu.sync_copy(zrows.at[pl.ds(0, GP)], pacc)
        pltpu.sync_copy(zflat.at[pl.ds(0, GP)], gcnt)

    pltpu.sync_copy(ones, ones_v)
    pltpu.sync_copy(batch_r.at[pl.ds(t * (NPT // 128), NPT // 128)], bat2d)
    plsc.subcore_barrier()

    def g_body(g, carry):
        node0 = t * NPT + g * 640

        @pl.when(c == 0)
        def _():
            pltpu.sync_copy(h2A.at[pl.ds(node0, 640)], rows_v)

        @pl.when(c == 1)
        def _():
            pltpu.sync_copy(h2B.at[pl.ds(node0, 640)], rows_v)

        for j in range(5):
            idxrow = bat2d.at[g * 5 + j]
            pltpu.sync_copy(rows_v.at[pl.ds(j * 128, 128)], pacc.at[idxrow],
                            add=True)
            pltpu.sync_copy(ones_v, gcnt.at[idxrow], add=True)
        return carry

    lax.fori_loop(0, NPT // 640, g_body, 0)
    plsc.subcore_barrier()

    @pl.when(t == 0)
    def _():
        pltpu.sync_copy(pacc.at[pl.ds(0, 512)], rows_v.at[pl.ds(0, 512)])
        pltpu.sync_copy(gcnt.at[pl.ds(0, 512)], cbuf.at[pl.ds(0, 512)])

        def r_body(r16, rc):
            base = r16 * 16
            iv = 1.0 / jnp.maximum(cbuf[pl.ds(base, 16)], 1.0)
            for i in range(16):
                s = iv[i]
                rows_v[base + i, pl.ds(0, 16)] = (
                    rows_v[base + i, pl.ds(0, 16)] * s)
                rows_v[base + i, pl.ds(16, 16)] = (
                    rows_v[base + i, pl.ds(16, 16)] * s)
            return rc

        lax.fori_loop(0, G // 16, r_body, 0)

        @pl.when(c == 0)
        def _():
            pltpu.sync_copy(rows_v.at[pl.ds(0, 512)], pA)

        @pl.when(c == 1)
        def _():
            pltpu.sync_copy(rows_v.at[pl.ds(0, 512)], pB)


BN = 1024


def _mm1_body(h_ref, wla, wlb, wra, wrb, a0A, a0B, r0A, r0B):
    h = h_ref[...]
    a0A[...] = jnp.dot(h, wla[...], preferred_element_type=_f32)
    a0B[...] = jnp.dot(h, wlb[...], preferred_element_type=_f32)
    r0A[...] = jnp.dot(h, wra[...], preferred_element_type=_f32)
    r0B[...] = jnp.dot(h, wrb[...], preferred_element_type=_f32)


_mm1 = pl.pallas_call(
    _mm1_body,
    grid=(NP // BN,),
    in_specs=[pl.BlockSpec((BN, EMB), lambda i: (i, 0))]
    + [pl.BlockSpec((EMB, 32), lambda i: (0, 0))] * 4,
    out_specs=[pl.BlockSpec((BN, 32), lambda i: (i, 0))] * 4,
    out_shape=[jax.ShapeDtypeStruct((NP, 32), _f32)] * 4,
)


def _mm2_body(mA, mB, r0A, r0B, b1A, b1B, qlAA, qlBA, qlAB, qlBB, qrAA, qrBA,
              qrAB, qrBB, a1A, a1B, r1A, r1B):
    h1A = jnp.maximum(mA[...] + r0A[...] + b1A[...], 0.0)
    h1B = jnp.maximum(mB[...] + r0B[...] + b1B[...], 0.0)
    a1A[...] = (jnp.dot(h1A, qlAA[...], preferred_element_type=_f32)
                + jnp.dot(h1B, qlBA[...], preferred_element_type=_f32))
    a1B[...] = (jnp.dot(h1A, qlAB[...], preferred_element_type=_f32)
                + jnp.dot(h1B, qlBB[...], preferred_element_type=_f32))
    r1A[...] = (jnp.dot(h1A, qrAA[...], preferred_element_type=_f32)
                + jnp.dot(h1B, qrBA[...], preferred_element_type=_f32))
    r1B[...] = (jnp.dot(h1A, qrAB[...], preferred_element_type=_f32)
                + jnp.dot(h1B, qrBB[...], preferred_element_type=_f32))


_mm2 = pl.pallas_call(
    _mm2_body,
    grid=(NP // BN,),
    in_specs=[pl.BlockSpec((BN, 32), lambda i: (i, 0))] * 4
    + [pl.BlockSpec((1, 32), lambda i: (0, 0))] * 2
    + [pl.BlockSpec((32, 32), lambda i: (0, 0))] * 8,
    out_specs=[pl.BlockSpec((BN, 32), lambda i: (i, 0))] * 4,
    out_shape=[jax.ShapeDtypeStruct((NP, 32), _f32)] * 4,
)


def _mm3_body(mA, mB, r1A, r1B, b2A, b2B, h2A, h2B):
    h2A[...] = jnp.maximum(mA[...] + r1A[...] + b2A[...], 0.0)
    h2B[...] = jnp.maximum(mB[...] + r1B[...] + b2B[...], 0.0)


_mm3 = pl.pallas_call(
    _mm3_body,
    grid=(NP // BN,),
    in_specs=[pl.BlockSpec((BN, 32), lambda i: (i, 0))] * 4
    + [pl.BlockSpec((1, 32), lambda i: (0, 0))] * 2,
    out_specs=[pl.BlockSpec((BN, 32), lambda i: (i, 0))] * 2,
    out_shape=[jax.ShapeDtypeStruct((NP, 32), _f32)] * 2,
)


def _fin_body(pA, pB, wlA, wlB, bl, out):
    out[...] = (jnp.dot(pA[...], wlA[...], preferred_element_type=_f32)
                + jnp.dot(pB[...], wlB[...], preferred_element_type=_f32)
                + bl[...])


_fin = pl.pallas_call(
    _fin_body,
    out_shape=jax.ShapeDtypeStruct((G, NCLS), _f32),
)


def kernel(x, edge_index, batch, emb_table, W1l, b1, W1r, W2l, b2, W2r, Wlin,
           blin):
    src = edge_index[0].astype(jnp.int32)
    dst = edge_index[1].astype(jnp.int32)
    xp = jnp.concatenate([x.astype(jnp.int32), jnp.zeros((NP - N,), jnp.int32)])
    x_r = xp.reshape(NP // 100, 100)
    srcr = jnp.concatenate([src, jnp.zeros((EP - E,), jnp.int32)]).reshape(
        EP // 128, 128)
    dstr = jnp.concatenate([dst, jnp.full((EP - E,), NP, jnp.int32)]).reshape(
        EP // 128, 128)
    batchr = jnp.concatenate(
        [batch.astype(jnp.int32), jnp.full((NP - N,), G, jnp.int32)]).reshape(
        NP // 128, 128)
    zrows = jnp.zeros((RPT, 32), _f32)
    zflat = jnp.zeros((RPT,), _f32)
    ones = jnp.ones((128,), _f32)

    h0 = _emb_gather(x_r, emb_table)
    a0A, a0B, r0A, r0B = _mm1(h0, W1l[:, :32], W1l[:, 32:], W1r[:, :32],
                              W1r[:, 32:])
    m1A, m1B, inv = _scatter1(srcr, dstr, a0A, a0B, zrows, zflat, ones)
    a1A, a1B, r1A, r1B = _mm2(
        m1A, m1B, r0A, r0B, b1[:32].reshape(1, 32), b1[32:].reshape(1, 32),
        W2l[:32, :32], W2l[32:, :32], W2l[:32, 32:], W2l[32:, 32:],
        W2r[:32, :32], W2r[32:, :32], W2r[:32, 32:], W2r[32:, 32:])
    m2A, m2B = _scatter2(srcr, dstr, a1A, a1B, inv, zrows, zflat, ones)
    h2A, h2B = _mm3(m2A, m2B, r1A, r1B, b2[:32].reshape(1, 32),
                    b2[32:].reshape(1, 32))
    pA, pB = _pool(h2A, h2B, batchr, zrows, zflat, ones)
    return _fin(pA, pB, Wlin[:32], Wlin[32:], blin.reshape(1, 2))
